# baseline (speedup 1.0000x reference)
import functools

import jax
import jax.numpy as jnp
from jax import lax
from jax.experimental import pallas as pl
from jax.experimental.pallas import tpu as pltpu
from jax.experimental.pallas import tpu_sc as plsc

D = 128
TOTAL_ROWS = 200000
NW = 32
ROWS_W = 6256
ROWS_LAST = TOTAL_ROWS - (NW - 1) * ROWS_W
CHUNK = 384
NCHUNK = 17
GROUPS = CHUNK // 16


def _sc_matvec_body(x_hbm, a2_hbm, y_hbm, buf0, buf1, a2_v, ybuf0, ybuf1,
                    sem0, sem1, osem0, osem1):
    cid = lax.axis_index("c")
    sid = lax.axis_index("s")
    wid = sid * 2 + cid
    start = wid * ROWS_W
    rows_w = jnp.where(wid == NW - 1, ROWS_LAST, ROWS_W)
    last_off = rows_w - CHUNK

    pltpu.sync_copy(a2_hbm, a2_v.at[pl.ds(0, D)])
    pltpu.sync_copy(a2_hbm.at[pl.ds(0, 16)], a2_v.at[pl.ds(D, 16)])

    iota16 = lax.iota(jnp.int32, 16)
    row_base = [(iota16 + (g * 16)) * D for g in range(GROUPS)]

    bufs = [buf0, buf1]
    ybufs = [ybuf0, ybuf1]
    sems = [sem0, sem1]
    osems = [osem0, osem1]
    offs = [jnp.minimum(i * CHUNK, last_off) for i in range(NCHUNK)]

    def start_in(i):
        row0 = start + offs[i]
        return pltpu.async_copy(
            x_hbm.at[pl.ds(row0 * D, CHUNK * D)], bufs[i % 2], sems[i % 2])

    in_cp = {0: start_in(0)}
    out_cp = {}
    for i in range(NCHUNK):
        p = i % 2
        if i + 1 < NCHUNK:
            in_cp[i + 1] = start_in(i + 1)
        in_cp[i].wait()

        def jbody(j, accs):
            feat = (iota16 + j) & (D - 1)
            coeff = a2_v[pl.ds(j, 16)]
            return tuple(
                accs[g] + plsc.load_gather(bufs[p], [row_base[g] + feat])
                * coeff
                for g in range(GROUPS)
            )

        zero = jnp.zeros((16,), jnp.float32)
        accs = lax.fori_loop(0, D, jbody, (zero,) * GROUPS)
        if i - 2 in out_cp:
            out_cp[i - 2].wait()
        for g in range(GROUPS):
            ybufs[p][pl.ds(g * 16, 16)] = accs[g]
        out_cp[i] = pltpu.async_copy(
            ybufs[p], y_hbm.at[pl.ds(start + offs[i], CHUNK)], osems[p])
    out_cp[NCHUNK - 2].wait()
    out_cp[NCHUNK - 1].wait()


@jax.jit
def _sc_matvec(x_flat, a2):
    mesh = plsc.VectorSubcoreMesh(core_axis_name="c", subcore_axis_name="s")
    return pl.kernel(
        _sc_matvec_body,
        out_type=jax.ShapeDtypeStruct((TOTAL_ROWS,), jnp.float32),
        mesh=mesh,
        compiler_params=pltpu.CompilerParams(needs_layout_passes=False),
        scratch_types=[
            pltpu.VMEM((CHUNK * D,), jnp.float32),
            pltpu.VMEM((CHUNK * D,), jnp.float32),
            pltpu.VMEM((D + 32,), jnp.float32),
            pltpu.VMEM((CHUNK,), jnp.float32),
            pltpu.VMEM((CHUNK,), jnp.float32),
            pltpu.SemaphoreType.DMA,
            pltpu.SemaphoreType.DMA,
            pltpu.SemaphoreType.DMA,
            pltpu.SemaphoreType.DMA,
        ],
    )(x_flat, a2)


def _norm_body(y_ref, tgt_ref, a1_ref, o_ref):
    c = jnp.sum(tgt_ref[...] * a1_ref[...], axis=1, keepdims=True)
    e = y_ref[...] + c
    e = jnp.where(e > 0, e, 0.01 * e)
    o_ref[...] = e / jnp.sum(e, axis=1, keepdims=True)


@jax.jit
def _tc_normalize(y, tgt, a1):
    batch, n = y.shape
    return pl.pallas_call(
        _norm_body,
        out_shape=jax.ShapeDtypeStruct((batch, n), jnp.float32),
    )(y, tgt, a1)


def kernel(x, node_index, adj_mask, a):
    batch, node_num, d = x.shape
    tgt = jnp.take(x, node_index, axis=1)
    a1 = a[:d, 0].reshape(1, d)
    a2 = a[d:, 0]
    y = _sc_matvec(x.reshape(batch * node_num * d), a2)
    return _tc_normalize(y.reshape(batch, node_num), tgt, a1)


# speedup vs baseline: 1.0768x; 1.0768x over previous
import jax
import jax.numpy as jnp
from jax import lax
from jax.experimental import pallas as pl
from jax.experimental.pallas import tpu as pltpu
from jax.experimental.pallas import tpu_sc as plsc

D = 128
TOTAL_ROWS = 200000
NW = 32
CHUNK = 256
SC_NCHUNK = 10
SC_ROWS_W = CHUNK * SC_NCHUNK
SC_ROWS = SC_ROWS_W * NW
TC_ROWS = TOTAL_ROWS - SC_ROWS
TC_BLOCK = 2048
GROUPS = CHUNK // 16


def _sc_matvec_body(x_hbm, a2_hbm, y_hbm, buf0, buf1, a2_v, ybuf0, ybuf1,
                    sem0, sem1, osem0, osem1):
    cid = lax.axis_index("c")
    sid = lax.axis_index("s")
    wid = sid * 2 + cid
    start = wid * SC_ROWS_W

    pltpu.sync_copy(a2_hbm, a2_v.at[pl.ds(0, D)])
    pltpu.sync_copy(a2_hbm.at[pl.ds(0, 16)], a2_v.at[pl.ds(D, 16)])

    iota16 = lax.iota(jnp.int32, 16)
    row_base = [(iota16 + (g * 16)) * D for g in range(GROUPS)]

    bufs = [buf0, buf1]
    ybufs = [ybuf0, ybuf1]
    sems = [sem0, sem1]
    osems = [osem0, osem1]

    def start_in(i):
        row0 = start + i * CHUNK
        return pltpu.async_copy(
            x_hbm.at[pl.ds(row0 * D, CHUNK * D)], bufs[i % 2], sems[i % 2])

    in_cp = {0: start_in(0)}
    out_cp = {}
    for i in range(SC_NCHUNK):
        p = i % 2
        if i + 1 < SC_NCHUNK:
            in_cp[i + 1] = start_in(i + 1)
        in_cp[i].wait()

        def jbody(j, accs):
            feat = (iota16 + j) & (D - 1)
            coeff = a2_v[pl.ds(j, 16)]
            return tuple(
                accs[g] + plsc.load_gather(bufs[p], [row_base[g] + feat])
                * coeff
                for g in range(GROUPS)
            )

        zero = jnp.zeros((16,), jnp.float32)
        accs = lax.fori_loop(0, D, jbody, (zero,) * GROUPS)
        if i - 2 in out_cp:
            out_cp[i - 2].wait()
        for g in range(GROUPS):
            ybufs[p][pl.ds(g * 16, 16)] = accs[g]
        out_cp[i] = pltpu.async_copy(
            ybufs[p], y_hbm.at[pl.ds(start + i * CHUNK, CHUNK)], osems[p])
    out_cp[SC_NCHUNK - 2].wait()
    out_cp[SC_NCHUNK - 1].wait()


def _sc_matvec(x_flat, a2):
    mesh = plsc.VectorSubcoreMesh(core_axis_name="c", subcore_axis_name="s")
    return pl.kernel(
        _sc_matvec_body,
        out_type=jax.ShapeDtypeStruct((SC_ROWS,), jnp.float32),
        mesh=mesh,
        compiler_params=pltpu.CompilerParams(needs_layout_passes=False),
        scratch_types=[
            pltpu.VMEM((CHUNK * D,), jnp.float32),
            pltpu.VMEM((CHUNK * D,), jnp.float32),
            pltpu.VMEM((D + 32,), jnp.float32),
            pltpu.VMEM((CHUNK,), jnp.float32),
            pltpu.VMEM((CHUNK,), jnp.float32),
            pltpu.SemaphoreType.DMA,
            pltpu.SemaphoreType.DMA,
            pltpu.SemaphoreType.DMA,
            pltpu.SemaphoreType.DMA,
        ],
    )(x_flat, a2)


def _tc_matvec_body(x_ref, a2_ref, o_ref):
    o_ref[...] = jnp.dot(
        x_ref[...], a2_ref[...], preferred_element_type=jnp.float32)[:, 0]


def _tc_matvec(x2d, a2col):
    grid = (TC_ROWS + TC_BLOCK - 1) // TC_BLOCK
    first = SC_ROWS // TC_BLOCK
    return pl.pallas_call(
        _tc_matvec_body,
        grid=(grid,),
        in_specs=[
            pl.BlockSpec((TC_BLOCK, D), lambda i: (first + i, 0)),
            pl.BlockSpec((D, 1), lambda i: (0, 0)),
        ],
        out_specs=pl.BlockSpec((TC_BLOCK,), lambda i: (i,)),
        out_shape=jax.ShapeDtypeStruct((TC_ROWS,), jnp.float32),
    )(x2d, a2col)


def _norm_body(y_ref, tgt_ref, a1_ref, o_ref):
    c = jnp.sum(tgt_ref[...] * a1_ref[...], axis=1, keepdims=True)
    e = y_ref[...] + c
    e = jnp.where(e > 0, e, 0.01 * e)
    o_ref[...] = e / jnp.sum(e, axis=1, keepdims=True)


def _tc_normalize(y, tgt, a1):
    batch, n = y.shape
    return pl.pallas_call(
        _norm_body,
        out_shape=jax.ShapeDtypeStruct((batch, n), jnp.float32),
    )(y, tgt, a1)


@jax.jit
def _run(x, node_index, a):
    batch, node_num, d = x.shape
    tgt = jnp.take(x, node_index, axis=1)
    a1 = a[:d, 0].reshape(1, d)
    a2 = a[d:, 0]
    y_sc = _sc_matvec(x.reshape(batch * node_num * d), a2)
    y_tc = _tc_matvec(x.reshape(batch * node_num, d), a2.reshape(d, 1))
    y = jnp.concatenate([y_sc, y_tc]).reshape(batch, node_num)
    return _tc_normalize(y, tgt, a1)


def kernel(x, node_index, adj_mask, a):
    return _run(x, node_index, a)


# speedup vs baseline: 1.3383x; 1.2429x over previous
import jax
import jax.numpy as jnp
from jax import lax
from jax.experimental import pallas as pl
from jax.experimental.pallas import tpu as pltpu
from jax.experimental.pallas import tpu_sc as plsc

D = 128
TOTAL_ROWS = 200000
NW = 32
CHUNK = 256
SC_NCHUNK = 10
SC_ROWS_W = CHUNK * SC_NCHUNK
SC_ROWS = SC_ROWS_W * NW
TC_ROWS = TOTAL_ROWS - SC_ROWS
TC_BLOCK = 2048
GROUPS = CHUNK // 16


def _sc_matvec_body(x_hbm, a2_hbm, y_hbm, buf0, buf1, a2_v, ybuf0, ybuf1,
                    sem0, sem1, osem0, osem1):
    cid = lax.axis_index("c")
    sid = lax.axis_index("s")
    wid = sid * 2 + cid
    start = wid * SC_ROWS_W

    pltpu.sync_copy(a2_hbm, a2_v.at[pl.ds(0, D)])
    pltpu.sync_copy(a2_hbm.at[pl.ds(0, 16)], a2_v.at[pl.ds(D, 16)])

    iota16 = lax.iota(jnp.int32, 16)
    row_base = [(iota16 + (g * 16)) * D for g in range(GROUPS)]

    bufs = [buf0, buf1]
    ybufs = [ybuf0, ybuf1]
    sems = [sem0, sem1]
    osems = [osem0, osem1]

    def start_in(i):
        row0 = start + i * CHUNK
        return pltpu.async_copy(
            x_hbm.at[pl.ds(row0 * D, CHUNK * D)], bufs[i % 2], sems[i % 2])

    in_cp = {0: start_in(0)}
    out_cp = {}
    for i in range(SC_NCHUNK):
        p = i % 2
        if i + 1 < SC_NCHUNK:
            in_cp[i + 1] = start_in(i + 1)
        in_cp[i].wait()

        def jbody(j, accs):
            feat = (iota16 + j) & (D - 1)
            coeff = a2_v[pl.ds(j, 16)]
            return tuple(
                accs[g] + plsc.load_gather(bufs[p], [row_base[g] + feat])
                * coeff
                for g in range(GROUPS)
            )

        zero = jnp.zeros((16,), jnp.float32)
        accs = lax.fori_loop(0, D, jbody, (zero,) * GROUPS)
        if i - 2 in out_cp:
            out_cp[i - 2].wait()
        for g in range(GROUPS):
            ybufs[p][pl.ds(g * 16, 16)] = accs[g]
        out_cp[i] = pltpu.async_copy(
            ybufs[p], y_hbm.at[pl.ds(start + i * CHUNK, CHUNK)], osems[p])
    out_cp[SC_NCHUNK - 2].wait()
    out_cp[SC_NCHUNK - 1].wait()


def _sc_matvec(x_flat, a2):
    mesh = plsc.VectorSubcoreMesh(core_axis_name="c", subcore_axis_name="s")
    return pl.kernel(
        _sc_matvec_body,
        out_type=jax.ShapeDtypeStruct((SC_ROWS,), jnp.float32),
        mesh=mesh,
        compiler_params=pltpu.CompilerParams(needs_layout_passes=False),
        scratch_types=[
            pltpu.VMEM((CHUNK * D,), jnp.float32),
            pltpu.VMEM((CHUNK * D,), jnp.float32),
            pltpu.VMEM((D + 32,), jnp.float32),
            pltpu.VMEM((CHUNK,), jnp.float32),
            pltpu.VMEM((CHUNK,), jnp.float32),
            pltpu.SemaphoreType.DMA,
            pltpu.SemaphoreType.DMA,
            pltpu.SemaphoreType.DMA,
            pltpu.SemaphoreType.DMA,
        ],
    )(x_flat, a2)


def _tc_matvec_body(x_ref, a2_ref, o_ref):
    o_ref[...] = lax.dot_general(
        a2_ref[...], x_ref[...], (((1,), (1,)), ((), ())),
        preferred_element_type=jnp.float32)


def _tc_matvec(x2d, a2row):
    grid = (TC_ROWS + TC_BLOCK - 1) // TC_BLOCK
    first = SC_ROWS // TC_BLOCK
    return pl.pallas_call(
        _tc_matvec_body,
        grid=(grid,),
        in_specs=[
            pl.BlockSpec((TC_BLOCK, D), lambda i: (first + i, 0)),
            pl.BlockSpec((1, D), lambda i: (0, 0)),
        ],
        out_specs=pl.BlockSpec((1, TC_BLOCK), lambda i: (0, i)),
        out_shape=jax.ShapeDtypeStruct((1, TC_ROWS), jnp.float32),
    )(x2d, a2row)


def _norm_body(y_ref, tgt_ref, a1_ref, o_ref):
    c = jnp.sum(tgt_ref[...] * a1_ref[...], axis=1, keepdims=True)
    e = y_ref[...] + c
    e = jnp.where(e > 0, e, 0.01 * e)
    o_ref[...] = e / jnp.sum(e, axis=1, keepdims=True)


def _tc_normalize(y, tgt, a1):
    batch, n = y.shape
    return pl.pallas_call(
        _norm_body,
        out_shape=jax.ShapeDtypeStruct((batch, n), jnp.float32),
    )(y, tgt, a1)


@jax.jit
def _run(x, node_index, a):
    batch, node_num, d = x.shape
    tgt = jnp.take(x, node_index, axis=1)
    a1 = a[:d, 0].reshape(1, d)
    a2 = a[d:, 0]
    y_sc = _sc_matvec(x.reshape(batch * node_num * d), a2)
    y_tc = _tc_matvec(x.reshape(batch * node_num, d), a2.reshape(1, d))
    y = jnp.concatenate([y_sc, y_tc.reshape(TC_ROWS)]).reshape(
        batch, node_num)
    return _tc_normalize(y, tgt, a1)


def kernel(x, node_index, adj_mask, a):
    return _run(x, node_index, a)


# speedup vs baseline: 1.3831x; 1.0335x over previous
import jax
import jax.numpy as jnp
from jax import lax
from jax.experimental import pallas as pl
from jax.experimental.pallas import tpu as pltpu
from jax.experimental.pallas import tpu_sc as plsc

D = 128
TOTAL_ROWS = 200000
NW = 32
CHUNK = 256
SC_ROWS = 100000
TC_ROWS = TOTAL_ROWS - SC_ROWS
ROWS_W = 3136
ROWS_LAST = SC_ROWS - (NW - 1) * ROWS_W
NCHUNK = 13
GROUPS = CHUNK // 16
TC_BLOCK = 2000


def _sc_matvec_body(x_hbm, a2_hbm, y_hbm, buf0, buf1, a2_v, ybuf0, ybuf1,
                    sem0, sem1, osem0, osem1):
    cid = lax.axis_index("c")
    sid = lax.axis_index("s")
    wid = sid * 2 + cid
    start = wid * ROWS_W
    rows_w = jnp.where(wid == NW - 1, ROWS_LAST, ROWS_W)
    last_off = rows_w - CHUNK

    pltpu.sync_copy(a2_hbm, a2_v.at[pl.ds(0, D)])
    pltpu.sync_copy(a2_hbm.at[pl.ds(0, 16)], a2_v.at[pl.ds(D, 16)])

    iota16 = lax.iota(jnp.int32, 16)
    row_base = [(iota16 + (g * 16)) * D for g in range(GROUPS)]

    bufs = [buf0, buf1]
    ybufs = [ybuf0, ybuf1]
    sems = [sem0, sem1]
    osems = [osem0, osem1]
    offs = [jnp.minimum(i * CHUNK, last_off) for i in range(NCHUNK)]

    def start_in(i):
        row0 = start + offs[i]
        return pltpu.async_copy(
            x_hbm.at[pl.ds(row0 * D, CHUNK * D)], bufs[i % 2], sems[i % 2])

    in_cp = {0: start_in(0)}
    out_cp = {}
    for i in range(NCHUNK):
        p = i % 2
        if i + 1 < NCHUNK:
            in_cp[i + 1] = start_in(i + 1)
        in_cp[i].wait()

        def jbody(j, accs):
            feat = (iota16 + j) & (D - 1)
            coeff = a2_v[pl.ds(j, 16)]
            return tuple(
                accs[g] + plsc.load_gather(bufs[p], [row_base[g] + feat])
                * coeff
                for g in range(GROUPS)
            )

        zero = jnp.zeros((16,), jnp.float32)
        accs = lax.fori_loop(0, D, jbody, (zero,) * GROUPS)
        if i - 2 in out_cp:
            out_cp[i - 2].wait()
        for g in range(GROUPS):
            ybufs[p][pl.ds(g * 16, 16)] = accs[g]
        out_cp[i] = pltpu.async_copy(
            ybufs[p], y_hbm.at[pl.ds(start + offs[i], CHUNK)], osems[p])
    out_cp[NCHUNK - 2].wait()
    out_cp[NCHUNK - 1].wait()


def _sc_matvec(x_flat, a2):
    mesh = plsc.VectorSubcoreMesh(core_axis_name="c", subcore_axis_name="s")
    return pl.kernel(
        _sc_matvec_body,
        out_type=jax.ShapeDtypeStruct((SC_ROWS,), jnp.float32),
        mesh=mesh,
        compiler_params=pltpu.CompilerParams(needs_layout_passes=False),
        scratch_types=[
            pltpu.VMEM((CHUNK * D,), jnp.float32),
            pltpu.VMEM((CHUNK * D,), jnp.float32),
            pltpu.VMEM((D + 32,), jnp.float32),
            pltpu.VMEM((CHUNK,), jnp.float32),
            pltpu.VMEM((CHUNK,), jnp.float32),
            pltpu.SemaphoreType.DMA,
            pltpu.SemaphoreType.DMA,
            pltpu.SemaphoreType.DMA,
            pltpu.SemaphoreType.DMA,
        ],
    )(x_flat, a2)


def _tc_matvec_body(x_ref, a2_ref, o_ref):
    o_ref[...] = lax.dot_general(
        a2_ref[...], x_ref[0], (((1,), (1,)), ((), ())),
        preferred_element_type=jnp.float32)[None]


def _tc_matvec(x3d, a2row):
    grid = TC_ROWS // TC_BLOCK
    first = SC_ROWS // TC_BLOCK
    return pl.pallas_call(
        _tc_matvec_body,
        grid=(grid,),
        in_specs=[
            pl.BlockSpec((1, TC_BLOCK, D), lambda i: (first + i, 0, 0)),
            pl.BlockSpec((1, D), lambda i: (0, 0)),
        ],
        out_specs=pl.BlockSpec((1, 1, TC_BLOCK), lambda i: (i, 0, 0)),
        out_shape=jax.ShapeDtypeStruct((grid, 1, TC_BLOCK), jnp.float32),
    )(x3d, a2row)


def _norm_body(y01_ref, y23_ref, tgt_ref, a1_ref, o_ref):
    y = jnp.concatenate([y01_ref[...], y23_ref[...]], axis=0)
    c = jnp.sum(tgt_ref[...] * a1_ref[...], axis=1, keepdims=True)
    e = y + c
    e = jnp.where(e > 0, e, 0.01 * e)
    o_ref[...] = e / jnp.sum(e, axis=1, keepdims=True)


def _tc_normalize(y01, y23, tgt, a1):
    return pl.pallas_call(
        _norm_body,
        out_shape=jax.ShapeDtypeStruct((4, 50000), jnp.float32),
    )(y01, y23, tgt, a1)


@jax.jit
def _run(x, node_index, a):
    batch, node_num, d = x.shape
    tgt = jnp.take(x, node_index, axis=1)
    a1 = a[:d, 0].reshape(1, d)
    a2 = a[d:, 0]
    y_sc = _sc_matvec(x.reshape(batch * node_num * d), a2)
    y_tc = _tc_matvec(
        x.reshape(TOTAL_ROWS // TC_BLOCK, TC_BLOCK, d), a2.reshape(1, d))
    return _tc_normalize(
        y_sc.reshape(2, node_num), y_tc.reshape(2, node_num), tgt, a1)


def kernel(x, node_index, adj_mask, a):
    return _run(x, node_index, a)


# speedup vs baseline: 1.4704x; 1.0631x over previous
import jax
import jax.numpy as jnp
from jax import lax
from jax.experimental import pallas as pl
from jax.experimental.pallas import tpu as pltpu
from jax.experimental.pallas import tpu_sc as plsc

D = 128
TOTAL_ROWS = 200000
NW = 32
CHUNK = 256
SC_ROWS = 100000
TC_ROWS = TOTAL_ROWS - SC_ROWS
ROWS_W = 3136
ROWS_LAST = SC_ROWS - (NW - 1) * ROWS_W
NCHUNK = 13
GROUPS = CHUNK // 16
TC_BLOCK = 2048


def _sc_matvec_body(x_hbm, a2_hbm, y_hbm, buf0, buf1, a2_v, ybuf0, ybuf1,
                    sem0, sem1, osem0, osem1):
    cid = lax.axis_index("c")
    sid = lax.axis_index("s")
    wid = sid * 2 + cid
    start = wid * ROWS_W
    rows_w = jnp.where(wid == NW - 1, ROWS_LAST, ROWS_W)
    last_off = rows_w - CHUNK

    pltpu.sync_copy(a2_hbm, a2_v.at[pl.ds(0, D)])
    pltpu.sync_copy(a2_hbm.at[pl.ds(0, 16)], a2_v.at[pl.ds(D, 16)])

    iota16 = lax.iota(jnp.int32, 16)
    row_base = [(iota16 + (g * 16)) * D for g in range(GROUPS)]

    bufs = [buf0, buf1]
    ybufs = [ybuf0, ybuf1]
    sems = [sem0, sem1]
    osems = [osem0, osem1]
    offs = [jnp.minimum(i * CHUNK, last_off) for i in range(NCHUNK)]

    def start_in(i):
        row0 = TC_ROWS + start + offs[i]
        return pltpu.async_copy(
            x_hbm.at[pl.ds(row0 * D, CHUNK * D)], bufs[i % 2], sems[i % 2])

    in_cp = {0: start_in(0)}
    out_cp = {}
    for i in range(NCHUNK):
        p = i % 2
        if i + 1 < NCHUNK:
            in_cp[i + 1] = start_in(i + 1)
        in_cp[i].wait()

        def jbody(j, accs):
            feat = (iota16 + j) & (D - 1)
            coeff = a2_v[pl.ds(j, 16)]
            return tuple(
                accs[g] + plsc.load_gather(bufs[p], [row_base[g] + feat])
                * coeff
                for g in range(GROUPS)
            )

        zero = jnp.zeros((16,), jnp.float32)
        accs = lax.fori_loop(0, D, jbody, (zero,) * GROUPS)
        if i - 2 in out_cp:
            out_cp[i - 2].wait()
        for g in range(GROUPS):
            ybufs[p][pl.ds(g * 16, 16)] = accs[g]
        out_cp[i] = pltpu.async_copy(
            ybufs[p], y_hbm.at[pl.ds(start + offs[i], CHUNK)], osems[p])
    out_cp[NCHUNK - 2].wait()
    out_cp[NCHUNK - 1].wait()


def _sc_matvec(x_flat, a2):
    mesh = plsc.VectorSubcoreMesh(core_axis_name="c", subcore_axis_name="s")
    return pl.kernel(
        _sc_matvec_body,
        out_type=jax.ShapeDtypeStruct((SC_ROWS,), jnp.float32),
        mesh=mesh,
        compiler_params=pltpu.CompilerParams(needs_layout_passes=False),
        scratch_types=[
            pltpu.VMEM((CHUNK * D,), jnp.float32),
            pltpu.VMEM((CHUNK * D,), jnp.float32),
            pltpu.VMEM((D + 32,), jnp.float32),
            pltpu.VMEM((CHUNK,), jnp.float32),
            pltpu.VMEM((CHUNK,), jnp.float32),
            pltpu.SemaphoreType.DMA,
            pltpu.SemaphoreType.DMA,
            pltpu.SemaphoreType.DMA,
            pltpu.SemaphoreType.DMA,
        ],
    )(x_flat, a2)


def _tc_matvec_body(x_ref, a2_ref, o_ref):
    o_ref[...] = lax.dot_general(
        a2_ref[...], x_ref[...], (((1,), (1,)), ((), ())),
        preferred_element_type=jnp.float32)[0]


def _tc_matvec(x2d, a2row):
    grid = (TC_ROWS + TC_BLOCK - 1) // TC_BLOCK
    return pl.pallas_call(
        _tc_matvec_body,
        grid=(grid,),
        in_specs=[
            pl.BlockSpec((TC_BLOCK, D), lambda i: (i, 0)),
            pl.BlockSpec((1, D), lambda i: (0, 0)),
        ],
        out_specs=pl.BlockSpec((TC_BLOCK,), lambda i: (i,)),
        out_shape=jax.ShapeDtypeStruct((TC_ROWS,), jnp.float32),
    )(x2d, a2row)


def _norm_body(y01_ref, y23_ref, tgt_ref, a1_ref, o_ref):
    c = tgt_ref[...] * a1_ref[...]
    rows = []
    for b in range(4):
        src = y01_ref if b < 2 else y23_ref
        seg = src[pl.ds((b % 2) * 50000, 50000)]
        e = seg + jnp.sum(c[b])
        e = jnp.where(e > 0, e, 0.01 * e)
        rows.append(e / jnp.sum(e))
    o_ref[...] = jnp.stack(rows, axis=0)


def _tc_normalize(y01, y23, tgt, a1):
    return pl.pallas_call(
        _norm_body,
        out_shape=jax.ShapeDtypeStruct((4, 50000), jnp.float32),
    )(y01, y23, tgt, a1)


@jax.jit
def _run(x, node_index, a):
    batch, node_num, d = x.shape
    tgt = jnp.take(x, node_index, axis=1)
    a1 = a[:d, 0].reshape(1, d)
    a2 = a[d:, 0]
    y_sc = _sc_matvec(x.reshape(batch * node_num * d), a2)
    y_tc = _tc_matvec(x.reshape(batch * node_num, d), a2.reshape(1, d))
    return _tc_normalize(y_tc, y_sc, tgt, a1)


def kernel(x, node_index, adj_mask, a):
    return _run(x, node_index, a)


# speedup vs baseline: 1.6412x; 1.1161x over previous
import jax
import jax.numpy as jnp
from jax import lax
from jax.experimental import pallas as pl
from jax.experimental.pallas import tpu as pltpu
from jax.experimental.pallas import tpu_sc as plsc

D = 128
TOTAL_ROWS = 200000
NW = 32
CHUNK = 256
SC_ROWS = 100000
TC_ROWS = TOTAL_ROWS - SC_ROWS
ROWS_W = 3136
ROWS_LAST = SC_ROWS - (NW - 1) * ROWS_W
NCHUNK = 13
GROUPS = CHUNK // 16
TC_BLOCK = 8192


def _sc_matvec_body(x_hbm, a2_hbm, y_hbm, buf0, buf1, a2_v, ybuf0, ybuf1,
                    sem0, sem1, osem0, osem1):
    cid = lax.axis_index("c")
    sid = lax.axis_index("s")
    wid = sid * 2 + cid
    start = wid * ROWS_W
    rows_w = jnp.where(wid == NW - 1, ROWS_LAST, ROWS_W)
    last_off = rows_w - CHUNK

    pltpu.sync_copy(a2_hbm, a2_v.at[pl.ds(0, D)])
    pltpu.sync_copy(a2_hbm.at[pl.ds(0, 16)], a2_v.at[pl.ds(D, 16)])

    iota16 = lax.iota(jnp.int32, 16)
    row_base = [(iota16 + (g * 16)) * D for g in range(GROUPS)]

    bufs = [buf0, buf1]
    ybufs = [ybuf0, ybuf1]
    sems = [sem0, sem1]
    osems = [osem0, osem1]
    offs = [jnp.minimum(i * CHUNK, last_off) for i in range(NCHUNK)]

    def start_in(i):
        row0 = TC_ROWS + start + offs[i]
        return pltpu.async_copy(
            x_hbm.at[pl.ds(row0 * D, CHUNK * D)], bufs[i % 2], sems[i % 2])

    in_cp = {0: start_in(0)}
    out_cp = {}
    for i in range(NCHUNK):
        p = i % 2
        if i + 1 < NCHUNK:
            in_cp[i + 1] = start_in(i + 1)
        in_cp[i].wait()

        def jbody(j, accs):
            feat = (iota16 + j) & (D - 1)
            coeff = a2_v[pl.ds(j, 16)]
            return tuple(
                accs[g] + plsc.load_gather(bufs[p], [row_base[g] + feat])
                * coeff
                for g in range(GROUPS)
            )

        zero = jnp.zeros((16,), jnp.float32)
        accs = lax.fori_loop(0, D, jbody, (zero,) * GROUPS)
        if i - 2 in out_cp:
            out_cp[i - 2].wait()
        for g in range(GROUPS):
            ybufs[p][pl.ds(g * 16, 16)] = accs[g]
        out_cp[i] = pltpu.async_copy(
            ybufs[p], y_hbm.at[pl.ds(start + offs[i], CHUNK)], osems[p])
    out_cp[NCHUNK - 2].wait()
    out_cp[NCHUNK - 1].wait()


def _sc_matvec(x_flat, a2):
    mesh = plsc.VectorSubcoreMesh(core_axis_name="c", subcore_axis_name="s")
    return pl.kernel(
        _sc_matvec_body,
        out_type=jax.ShapeDtypeStruct((SC_ROWS,), jnp.float32),
        mesh=mesh,
        compiler_params=pltpu.CompilerParams(needs_layout_passes=False),
        scratch_types=[
            pltpu.VMEM((CHUNK * D,), jnp.float32),
            pltpu.VMEM((CHUNK * D,), jnp.float32),
            pltpu.VMEM((D + 32,), jnp.float32),
            pltpu.VMEM((CHUNK,), jnp.float32),
            pltpu.VMEM((CHUNK,), jnp.float32),
            pltpu.SemaphoreType.DMA,
            pltpu.SemaphoreType.DMA,
            pltpu.SemaphoreType.DMA,
            pltpu.SemaphoreType.DMA,
        ],
    )(x_flat, a2)


def _tc_matvec_body(x_ref, a2_ref, o_ref):
    o_ref[...] = lax.dot_general(
        a2_ref[...], x_ref[...], (((1,), (1,)), ((), ())),
        preferred_element_type=jnp.float32)[0]


def _tc_matvec(x2d, a2row):
    grid = (TC_ROWS + TC_BLOCK - 1) // TC_BLOCK
    return pl.pallas_call(
        _tc_matvec_body,
        grid=(grid,),
        in_specs=[
            pl.BlockSpec((TC_BLOCK, D), lambda i: (i, 0)),
            pl.BlockSpec((1, D), lambda i: (0, 0)),
        ],
        out_specs=pl.BlockSpec((TC_BLOCK,), lambda i: (i,)),
        out_shape=jax.ShapeDtypeStruct((TC_ROWS,), jnp.float32),
    )(x2d, a2row)


def _norm_body(y01_ref, y23_ref, tgt_ref, a1_ref, o_ref):
    c = tgt_ref[...] * a1_ref[...]
    rows = []
    for b in range(4):
        src = y01_ref if b < 2 else y23_ref
        seg = src[pl.ds((b % 2) * 50000, 50000)]
        e = seg + jnp.sum(c[b])
        e = jnp.where(e > 0, e, 0.01 * e)
        rows.append(e / jnp.sum(e))
    o_ref[...] = jnp.stack(rows, axis=0)


def _tc_normalize(y01, y23, tgt, a1):
    return pl.pallas_call(
        _norm_body,
        out_shape=jax.ShapeDtypeStruct((4, 50000), jnp.float32),
    )(y01, y23, tgt, a1)


@jax.jit
def _run(x, node_index, a):
    batch, node_num, d = x.shape
    tgt = jnp.take(x, node_index, axis=1)
    a1 = a[:d, 0].reshape(1, d)
    a2 = a[d:, 0]
    y_sc = _sc_matvec(x.reshape(batch * node_num * d), a2)
    y_tc = _tc_matvec(x.reshape(batch * node_num, d), a2.reshape(1, d))
    return _tc_normalize(y_tc, y_sc, tgt, a1)


def kernel(x, node_index, adj_mask, a):
    return _run(x, node_index, a)


# speedup vs baseline: 1.6467x; 1.0034x over previous
import jax
import jax.numpy as jnp
from jax import lax
from jax.experimental import pallas as pl
from jax.experimental.pallas import tpu as pltpu
from jax.experimental.pallas import tpu_sc as plsc

D = 128
TOTAL_ROWS = 200000
NW = 32
CHUNK = 256
SC_ROWS = 100000
TC_ROWS = TOTAL_ROWS - SC_ROWS
ROWS_W = 3136
ROWS_LAST = SC_ROWS - (NW - 1) * ROWS_W
NCHUNK = 13
GROUPS = CHUNK // 16
TC_BLOCK = 16384


def _sc_matvec_body(x_hbm, a2_hbm, y_hbm, buf0, buf1, a2_v, ybuf0, ybuf1,
                    sem0, sem1, osem0, osem1):
    cid = lax.axis_index("c")
    sid = lax.axis_index("s")
    wid = sid * 2 + cid
    start = wid * ROWS_W
    rows_w = jnp.where(wid == NW - 1, ROWS_LAST, ROWS_W)
    last_off = rows_w - CHUNK

    pltpu.sync_copy(a2_hbm, a2_v.at[pl.ds(0, D)])
    pltpu.sync_copy(a2_hbm.at[pl.ds(0, 16)], a2_v.at[pl.ds(D, 16)])

    iota16 = lax.iota(jnp.int32, 16)
    row_base = [(iota16 + (g * 16)) * D for g in range(GROUPS)]

    bufs = [buf0, buf1]
    ybufs = [ybuf0, ybuf1]
    sems = [sem0, sem1]
    osems = [osem0, osem1]
    offs = [jnp.minimum(i * CHUNK, last_off) for i in range(NCHUNK)]

    def start_in(i):
        row0 = TC_ROWS + start + offs[i]
        return pltpu.async_copy(
            x_hbm.at[pl.ds(row0 * D, CHUNK * D)], bufs[i % 2], sems[i % 2])

    in_cp = {0: start_in(0)}
    out_cp = {}
    for i in range(NCHUNK):
        p = i % 2
        if i + 1 < NCHUNK:
            in_cp[i + 1] = start_in(i + 1)
        in_cp[i].wait()

        def jbody(j, accs):
            feat = (iota16 + j) & (D - 1)
            coeff = a2_v[pl.ds(j, 16)]
            return tuple(
                accs[g] + plsc.load_gather(bufs[p], [row_base[g] + feat])
                * coeff
                for g in range(GROUPS)
            )

        zero = jnp.zeros((16,), jnp.float32)
        accs = lax.fori_loop(0, D, jbody, (zero,) * GROUPS)
        if i - 2 in out_cp:
            out_cp[i - 2].wait()
        for g in range(GROUPS):
            ybufs[p][pl.ds(g * 16, 16)] = accs[g]
        out_cp[i] = pltpu.async_copy(
            ybufs[p], y_hbm.at[pl.ds(start + offs[i], CHUNK)], osems[p])
    out_cp[NCHUNK - 2].wait()
    out_cp[NCHUNK - 1].wait()


def _sc_matvec(x_flat, a2):
    mesh = plsc.VectorSubcoreMesh(core_axis_name="c", subcore_axis_name="s")
    return pl.kernel(
        _sc_matvec_body,
        out_type=jax.ShapeDtypeStruct((SC_ROWS,), jnp.float32),
        mesh=mesh,
        compiler_params=pltpu.CompilerParams(needs_layout_passes=False),
        scratch_types=[
            pltpu.VMEM((CHUNK * D,), jnp.float32),
            pltpu.VMEM((CHUNK * D,), jnp.float32),
            pltpu.VMEM((D + 32,), jnp.float32),
            pltpu.VMEM((CHUNK,), jnp.float32),
            pltpu.VMEM((CHUNK,), jnp.float32),
            pltpu.SemaphoreType.DMA,
            pltpu.SemaphoreType.DMA,
            pltpu.SemaphoreType.DMA,
            pltpu.SemaphoreType.DMA,
        ],
    )(x_flat, a2)


def _tc_matvec_body(x_ref, a2_ref, o_ref):
    o_ref[...] = lax.dot_general(
        a2_ref[...], x_ref[...], (((1,), (1,)), ((), ())),
        preferred_element_type=jnp.float32)[0]


def _tc_matvec(x2d, a2row):
    grid = (TC_ROWS + TC_BLOCK - 1) // TC_BLOCK
    return pl.pallas_call(
        _tc_matvec_body,
        grid=(grid,),
        in_specs=[
            pl.BlockSpec((TC_BLOCK, D), lambda i: (i, 0)),
            pl.BlockSpec((1, D), lambda i: (0, 0)),
        ],
        out_specs=pl.BlockSpec((TC_BLOCK,), lambda i: (i,)),
        out_shape=jax.ShapeDtypeStruct((TC_ROWS,), jnp.float32),
    )(x2d, a2row)


def _norm_body(y01_ref, y23_ref, tgt_ref, a1_ref, o_ref):
    c = tgt_ref[...] * a1_ref[...]
    rows = []
    for b in range(4):
        src = y01_ref if b < 2 else y23_ref
        seg = src[pl.ds((b % 2) * 50000, 50000)]
        e = seg + jnp.sum(c[b])
        e = jnp.where(e > 0, e, 0.01 * e)
        rows.append(e / jnp.sum(e))
    o_ref[...] = jnp.stack(rows, axis=0)


def _tc_normalize(y01, y23, tgt, a1):
    return pl.pallas_call(
        _norm_body,
        out_shape=jax.ShapeDtypeStruct((4, 50000), jnp.float32),
    )(y01, y23, tgt, a1)


@jax.jit
def _run(x, node_index, a):
    batch, node_num, d = x.shape
    tgt = jnp.take(x, node_index, axis=1)
    a1 = a[:d, 0].reshape(1, d)
    a2 = a[d:, 0]
    y_sc = _sc_matvec(x.reshape(batch * node_num * d), a2)
    y_tc = _tc_matvec(x.reshape(batch * node_num, d), a2.reshape(1, d))
    return _tc_normalize(y_tc, y_sc, tgt, a1)


def kernel(x, node_index, adj_mask, a):
    return _run(x, node_index, a)
